# baseline (speedup 1.0000x reference)
import jax
import jax.numpy as jnp
from jax import lax
from jax.experimental import pallas as pl
from jax.experimental.pallas import tpu as pltpu
from jax.experimental.pallas import tpu_sc as plsc

B = 1024
L = 200
D = 128
N = B * L
NC, NS = 2, 16
NW = NC * NS

K = 4
CB = B // K
SLICE = CB * L
PER_W = SLICE // NW
C = 64
NCHUNK = PER_W // C
NB = 5

BB = 32


def _positional_encoding():
    pos = jnp.arange(L, dtype=jnp.float32)[:, None]
    i = jnp.arange(0, D, 2, dtype=jnp.float32)
    div = jnp.exp(-jnp.log(10000.0) * i / D)
    pe = jnp.zeros((L, D), dtype=jnp.float32)
    pe = pe.at[:, 0::2].set(jnp.sin(pos * div))
    pe = pe.at[:, 1::2].set(jnp.cos(pos * div))
    return pe


def _gather_body(tok_hbm, table_hbm, out_hbm, *scr):
    idx = scr[0:NB]
    rows = scr[NB:2 * NB]
    si = scr[2 * NB:3 * NB]
    sg = scr[3 * NB:4 * NB]
    so = scr[4 * NB:5 * NB]

    wid = lax.axis_index("s") * NC + lax.axis_index("c")
    base = wid * PER_W

    def idx_start(c, b):
        pltpu.async_copy(tok_hbm.at[pl.ds(base + c * C, C)], idx[b], si[b])

    def idx_wait(b):
        pltpu.make_async_copy(tok_hbm.at[pl.ds(base, C)], idx[b], si[b]).wait()

    def gather_start(b):
        pltpu.async_copy(table_hbm.at[idx[b]], rows[b], sg[b])

    def gather_wait(b):
        pltpu.make_async_copy(table_hbm.at[idx[b]], rows[b], sg[b]).wait()

    def out_start(c, b):
        pltpu.async_copy(rows[b], out_hbm.at[pl.ds(base + c * C, C)], so[b])

    def out_wait(b):
        pltpu.make_async_copy(rows[b], out_hbm.at[pl.ds(base, C)], so[b]).wait()

    idx_start(0, 0)
    idx_start(1, 1)
    idx_start(2, 2)
    idx_wait(0)
    gather_start(0)

    def group(g, carry):
        for b in range(NB):
            c = g * NB + b
            s1 = (b + 1) % NB

            @pl.when(c + 1 < NCHUNK)
            def _fire_gather():
                @pl.when(c + 1 >= NB)
                def _drain_out():
                    out_wait(s1)

                idx_wait(s1)
                gather_start(s1)

            @pl.when(c + 3 < NCHUNK)
            def _prefetch_idx():
                idx_start(c + 3, (b + 3) % NB)

            gather_wait(b)
            out_start(c, b)
        return carry

    lax.fori_loop(0, NCHUNK // NB, group, 0)
    for b in range(NB):
        out_wait(b)


def _sc_gather(tok_slice, table):
    mesh = plsc.VectorSubcoreMesh(core_axis_name="c", subcore_axis_name="s")
    scratch = (
        [pltpu.VMEM((C,), jnp.int32) for _ in range(NB)]
        + [pltpu.VMEM((C, D), jnp.float32) for _ in range(NB)]
        + [pltpu.SemaphoreType.DMA for _ in range(3 * NB)]
    )
    return pl.kernel(
        _gather_body,
        mesh=mesh,
        out_type=jax.ShapeDtypeStruct((SLICE, D), jnp.float32),
        scratch_types=scratch,
    )(tok_slice, table)


def _add_body(raw_ref, pe_ref, o_ref):
    o_ref[...] = raw_ref[...] + pe_ref[...]


def _tc_add(raw3, pe3):
    return pl.pallas_call(
        _add_body,
        grid=(CB // BB,),
        in_specs=[
            pl.BlockSpec((BB, L, D), lambda i: (i, 0, 0)),
            pl.BlockSpec((1, L, D), lambda i: (0, 0, 0)),
        ],
        out_specs=pl.BlockSpec((BB, L, D), lambda i: (i, 0, 0)),
        out_shape=jax.ShapeDtypeStruct((CB, L, D), jnp.float32),
    )(raw3, pe3)


def kernel(tokens, table):
    pe3 = _positional_encoding()[None]
    outs = []
    for i in range(K):
        tok_i = tokens[i * CB:(i + 1) * CB].reshape(SLICE)
        raw = _sc_gather(tok_i, table)
        outs.append(_tc_add(raw.reshape(CB, L, D), pe3))
    return jnp.concatenate(outs, axis=0)
